# Initial kernel scaffold; baseline (speedup 1.0000x reference)
import jax
import jax.numpy as jnp
from jax.experimental import pallas as pl


def kernel(input_ids, token_type_ids, age, abspos, concept_table, segment_table, age_w0, age_phi0, age_w, age_phi, abspos_w0, abspos_phi0, abspos_w, abspos_phi, ln_gamma, ln_beta):
    raise NotImplementedError("write your pallas kernel here")


# speedup vs baseline: 1.7529x; 1.7529x over previous
import jax
import jax.numpy as jnp
from jax.experimental import pallas as pl
from jax.experimental.pallas import tpu as pltpu
from jax.experimental.pallas import tpu_sc as plsc

HID = 64
EPS = 1e-12
CLIP_MIN = -100.0
CLIP_MAX = 100.0

_GATHER_WINDOW = 256
_TC_ROWS = 4096


def _sc_gather(table_pairs, ids_half, n):
    mesh = plsc.VectorSubcoreMesh(core_axis_name="core", subcore_axis_name="subcore")
    ids2d = ids_half.reshape(1, n)

    @pl.kernel(
        out_type=jax.ShapeDtypeStruct((n, 2 * HID), table_pairs.dtype),
        mesh=mesh,
    )
    def gather_kernel(table_hbm, ids_hbm, out_hbm):
        def body(i_vmem, o_vmem):
            pltpu.sync_copy(table_hbm.at[i_vmem.at[0]], o_vmem)

        pltpu.emit_pipeline(
            body,
            grid=(n // _GATHER_WINDOW,),
            in_specs=[pl.BlockSpec((1, _GATHER_WINDOW), lambda i: (0, i))],
            out_specs=[pl.BlockSpec((_GATHER_WINDOW, 2 * HID), lambda i: (i, 0))],
            core_axis_name=("core", "subcore"),
            dimension_semantics=(pltpu.PARALLEL,),
        )(ids_hbm, out_hbm)

    return gather_kernel(table_pairs, ids2d)


def _combine_body(g_ref, enc_ref, age_ref, abspos_ref, par_ref, out_ref):
    r = g_ref.shape[0]
    enc = enc_ref[...].reshape(r, 1)
    parity = enc >= 2.0
    ttf = enc - jnp.where(parity, 2.0, 0.0)
    g = jnp.where(parity, g_ref[:, HID:], g_ref[:, :HID])
    w_age = par_ref[0:1, :]
    phi_age = par_ref[1:2, :]
    w_abs = par_ref[2:3, :]
    phi_abs = par_ref[3:4, :]
    gamma = par_ref[4:5, :]
    beta = par_ref[5:6, :]
    seg0 = par_ref[6:7, :]
    seg1 = par_ref[7:8, :]

    age = age_ref[...].reshape(r, 1)
    abspos = abspos_ref[...].reshape(r, 1)

    lane = jax.lax.broadcasted_iota(jnp.int32, (1, HID), 1)
    val_a = age * w_age + phi_age
    t2v_a = jnp.where(lane == 0, jnp.clip(val_a, CLIP_MIN, CLIP_MAX), jnp.cos(val_a))
    val_b = abspos * w_abs + phi_abs
    t2v_b = jnp.where(lane == 0, jnp.clip(val_b, CLIP_MIN, CLIP_MAX), jnp.cos(val_b))

    emb = g + seg0 + ttf * (seg1 - seg0) + t2v_a + t2v_b
    mu = jnp.mean(emb, axis=1, keepdims=True)
    c = emb - mu
    var = jnp.mean(c * c, axis=1, keepdims=True)
    out_ref[...] = c * jax.lax.rsqrt(var + EPS) * gamma + beta


def _tc_combine(gathered, enc, age_f, abspos_f, params, n):
    grid = n // _TC_ROWS
    return pl.pallas_call(
        _combine_body,
        grid=(grid,),
        in_specs=[
            pl.BlockSpec((_TC_ROWS, 2 * HID), lambda i: (i, 0)),
            pl.BlockSpec((_TC_ROWS,), lambda i: (i,)),
            pl.BlockSpec((_TC_ROWS,), lambda i: (i,)),
            pl.BlockSpec((_TC_ROWS,), lambda i: (i,)),
            pl.BlockSpec((8, HID), lambda i: (0, 0)),
        ],
        out_specs=pl.BlockSpec((_TC_ROWS, HID), lambda i: (i, 0)),
        out_shape=jax.ShapeDtypeStruct((n, HID), jnp.float32),
    )(gathered, enc, age_f, abspos_f, params)


def kernel(input_ids, token_type_ids, age, abspos, concept_table, segment_table,
           age_w0, age_phi0, age_w, age_phi,
           abspos_w0, abspos_phi0, abspos_w, abspos_phi,
           ln_gamma, ln_beta):
    b, s = input_ids.shape
    n = b * s

    w_cat_age = jnp.concatenate([age_w0.reshape(1), age_w.reshape(HID - 1)])
    phi_cat_age = jnp.concatenate([age_phi0.reshape(1), age_phi.reshape(HID - 1)])
    w_cat_abs = jnp.concatenate([abspos_w0.reshape(1), abspos_w.reshape(HID - 1)])
    phi_cat_abs = jnp.concatenate([abspos_phi0.reshape(1), abspos_phi.reshape(HID - 1)])
    params = jnp.stack([
        w_cat_age, phi_cat_age, w_cat_abs, phi_cat_abs,
        ln_gamma, ln_beta, segment_table[0], segment_table[1],
    ])

    ids_flat = input_ids.reshape(n)
    table_pairs = concept_table.reshape(-1, 2 * HID)
    gathered = _sc_gather(table_pairs, ids_flat >> 1, n)
    enc = (token_type_ids.reshape(n) + ((ids_flat & 1) << 1)).astype(jnp.float32)
    out = _tc_combine(
        gathered,
        enc,
        age.reshape(n),
        abspos.reshape(n),
        params,
        n,
    )
    return out.reshape(b, s, HID)


# speedup vs baseline: 3.5507x; 2.0256x over previous
import jax
import jax.numpy as jnp
from jax.experimental import pallas as pl
from jax.experimental.pallas import tpu as pltpu
from jax.experimental.pallas import tpu_sc as plsc

HID = 64
EPS = 1e-12
CLIP_MIN = -100.0
CLIP_MAX = 100.0

_GATHER_WINDOW = 256
_TC_ROWS = 4096

_INV_2PI = 0.15915494309189535
_CW1 = 6.28125
_CW2 = 0.0019350052
_CW3 = 3.019916e-07
_COS_C = (0.99999917, -0.49999392, 0.04165949, -0.0013857835,
          2.4190384e-05, -2.191602e-07)


def _fast_cos(x):
    k = jnp.round(x * _INV_2PI)
    r = x - k * _CW1
    r = r - k * _CW2
    r = r - k * _CW3
    t = r * r
    acc = jnp.full_like(t, _COS_C[5])
    for c in _COS_C[4::-1]:
        acc = acc * t + c
    return acc


def _sc_gather(table_pairs, ids_half, n):
    mesh = plsc.VectorSubcoreMesh(core_axis_name="core", subcore_axis_name="subcore")
    ids2d = ids_half.reshape(1, n)

    @pl.kernel(
        out_type=jax.ShapeDtypeStruct((n, 2 * HID), table_pairs.dtype),
        mesh=mesh,
    )
    def gather_kernel(table_hbm, ids_hbm, out_hbm):
        def body(i_vmem, o_vmem):
            pltpu.sync_copy(table_hbm.at[i_vmem.at[0]], o_vmem)

        pltpu.emit_pipeline(
            body,
            grid=(n // _GATHER_WINDOW,),
            in_specs=[pl.BlockSpec((1, _GATHER_WINDOW), lambda i: (0, i))],
            out_specs=[pl.BlockSpec((_GATHER_WINDOW, 2 * HID), lambda i: (i, 0))],
            core_axis_name=("core", "subcore"),
            dimension_semantics=(pltpu.PARALLEL,),
        )(ids_hbm, out_hbm)

    return gather_kernel(table_pairs, ids2d)


def _combine_body(g_ref, enc_ref, age_ref, abspos_ref, par_ref, out_ref):
    r = g_ref.shape[0]
    enc = enc_ref[...].reshape(r, 1)
    parity = enc >= 2.0
    ttf = enc - jnp.where(parity, 2.0, 0.0)
    g = jnp.where(parity, g_ref[:, HID:], g_ref[:, :HID])
    w_age = par_ref[0:1, :]
    phi_age = par_ref[1:2, :]
    w_abs = par_ref[2:3, :]
    phi_abs = par_ref[3:4, :]
    gamma = par_ref[4:5, :]
    beta = par_ref[5:6, :]
    seg0 = par_ref[6:7, :]
    seg1 = par_ref[7:8, :]

    age = age_ref[...].reshape(r, 1)
    abspos = abspos_ref[...].reshape(r, 1)

    lane = jax.lax.broadcasted_iota(jnp.int32, (1, HID), 1)
    val_a = age * w_age + phi_age
    t2v_a = jnp.where(lane == 0, jnp.clip(val_a, CLIP_MIN, CLIP_MAX), _fast_cos(val_a))
    val_b = abspos * w_abs + phi_abs
    t2v_b = jnp.where(lane == 0, jnp.clip(val_b, CLIP_MIN, CLIP_MAX), _fast_cos(val_b))

    emb = g + seg0 + ttf * (seg1 - seg0) + t2v_a + t2v_b
    mu = jnp.mean(emb, axis=1, keepdims=True)
    c = emb - mu
    var = jnp.mean(c * c, axis=1, keepdims=True)
    out_ref[...] = c * jax.lax.rsqrt(var + EPS) * gamma + beta


def _tc_combine(gathered, enc, age_f, abspos_f, params, n):
    grid = n // _TC_ROWS
    return pl.pallas_call(
        _combine_body,
        grid=(grid,),
        in_specs=[
            pl.BlockSpec((_TC_ROWS, 2 * HID), lambda i: (i, 0)),
            pl.BlockSpec((_TC_ROWS,), lambda i: (i,)),
            pl.BlockSpec((_TC_ROWS,), lambda i: (i,)),
            pl.BlockSpec((_TC_ROWS,), lambda i: (i,)),
            pl.BlockSpec((8, HID), lambda i: (0, 0)),
        ],
        out_specs=pl.BlockSpec((_TC_ROWS, HID), lambda i: (i, 0)),
        out_shape=jax.ShapeDtypeStruct((n, HID), jnp.float32),
        compiler_params=pltpu.CompilerParams(dimension_semantics=("parallel",)),
    )(gathered, enc, age_f, abspos_f, params)


def kernel(input_ids, token_type_ids, age, abspos, concept_table, segment_table,
           age_w0, age_phi0, age_w, age_phi,
           abspos_w0, abspos_phi0, abspos_w, abspos_phi,
           ln_gamma, ln_beta):
    b, s = input_ids.shape
    n = b * s

    w_cat_age = jnp.concatenate([age_w0.reshape(1), age_w.reshape(HID - 1)])
    phi_cat_age = jnp.concatenate([age_phi0.reshape(1), age_phi.reshape(HID - 1)])
    w_cat_abs = jnp.concatenate([abspos_w0.reshape(1), abspos_w.reshape(HID - 1)])
    phi_cat_abs = jnp.concatenate([abspos_phi0.reshape(1), abspos_phi.reshape(HID - 1)])
    params = jnp.stack([
        w_cat_age, phi_cat_age, w_cat_abs, phi_cat_abs,
        ln_gamma, ln_beta, segment_table[0], segment_table[1],
    ])

    ids_flat = input_ids.reshape(n)
    table_pairs = concept_table.reshape(-1, 2 * HID)
    gathered = _sc_gather(table_pairs, ids_flat >> 1, n)
    enc = (token_type_ids.reshape(n) + ((ids_flat & 1) << 1)).astype(jnp.float32)
    out = _tc_combine(
        gathered,
        enc,
        age.reshape(n),
        abspos.reshape(n),
        params,
        n,
    )
    return out.reshape(b, s, HID)


# speedup vs baseline: 5.4996x; 1.5489x over previous
import jax
import jax.numpy as jnp
from jax.experimental import pallas as pl
from jax.experimental.pallas import tpu as pltpu
from jax.experimental.pallas import tpu_sc as plsc

HID = 64
EPS = 1e-12
CLIP_MIN = -100.0
CLIP_MAX = 100.0

_GATHER_WINDOW = 256
_TC_ROWS = 4096

_INV_2PI = 0.15915494309189535
_CW1 = 6.28125
_CW2 = 0.0019350052
_COS_C = (0.99995601, -0.49977785, 0.041486766, -0.0013375250, 1.8691693e-05)


def _fast_cos(x):
    k = jax.lax.round(x * _INV_2PI, jax.lax.RoundingMethod.TO_NEAREST_EVEN)
    r = x - k * _CW1
    r = r - k * _CW2
    t = r * r
    acc = jnp.full_like(t, _COS_C[4])
    for c in _COS_C[3::-1]:
        acc = acc * t + c
    return acc


def _sc_gather(table_pairs, ids_half, n):
    mesh = plsc.VectorSubcoreMesh(core_axis_name="core", subcore_axis_name="subcore")
    ids2d = ids_half.reshape(1, n)

    @pl.kernel(
        out_type=jax.ShapeDtypeStruct((n, 2 * HID), table_pairs.dtype),
        mesh=mesh,
    )
    def gather_kernel(table_hbm, ids_hbm, out_hbm):
        def body(i_vmem, o_vmem):
            pltpu.sync_copy(table_hbm.at[i_vmem.at[0]], o_vmem)

        pltpu.emit_pipeline(
            body,
            grid=(n // _GATHER_WINDOW,),
            in_specs=[pl.BlockSpec((1, _GATHER_WINDOW), lambda i: (0, i))],
            out_specs=[pl.BlockSpec((_GATHER_WINDOW, 2 * HID), lambda i: (i, 0))],
            core_axis_name=("core", "subcore"),
            dimension_semantics=(pltpu.PARALLEL,),
        )(ids_hbm, out_hbm)

    return gather_kernel(table_pairs, ids2d)


def _combine_body(g_ref, enc_ref, age_ref, abspos_ref, par_ref, out_ref):
    p = g_ref.shape[0] // 2
    encc = enc_ref[...].reshape(2 * p, 1)
    agec = age_ref[...].reshape(2 * p, 1)
    absc = abspos_ref[...].reshape(2 * p, 1)
    lane = jax.lax.broadcasted_iota(jnp.int32, (1, 2 * HID), 1)
    low = lane < HID

    encA, encB = encc[0:p], encc[p:2 * p]
    enc_pk = jnp.where(low, encA, encB)
    age_pk = jnp.where(low, agec[0:p], agec[p:2 * p])
    abs_pk = jnp.where(low, absc[0:p], absc[p:2 * p])

    par_pk = enc_pk >= 2.0
    ttf_pk = enc_pk - jnp.where(par_pk, 2.0, 0.0)
    parA = encA >= 2.0
    parB = encB >= 2.0

    gA = g_ref[0:p, :]
    gB = g_ref[p:2 * p, :]
    gAr = pltpu.roll(gA, HID, 1)
    gBr = pltpu.roll(gB, HID, 1)
    g_pk = jnp.where(low, jnp.where(parA, gAr, gA), jnp.where(parB, gB, gBr))

    w_age = par_ref[0:1, :]
    phi_age = par_ref[1:2, :]
    w_abs = par_ref[2:3, :]
    phi_abs = par_ref[3:4, :]
    gamma = par_ref[4:5, :]
    beta = par_ref[5:6, :]
    seg0 = par_ref[6:7, :]
    seg1 = par_ref[7:8, :]

    m0 = (lane == 0) | (lane == HID)
    val_a = age_pk * w_age + phi_age
    t2v_a = jnp.where(m0, jnp.clip(val_a, CLIP_MIN, CLIP_MAX), _fast_cos(val_a))
    val_b = abs_pk * w_abs + phi_abs
    t2v_b = jnp.where(m0, jnp.clip(val_b, CLIP_MIN, CLIP_MAX), _fast_cos(val_b))

    emb = g_pk + seg0 + ttf_pk * (seg1 - seg0) + t2v_a + t2v_b

    sgn = jnp.where(low, 1.0, -1.0)
    t_sum = jnp.sum(emb, axis=1, keepdims=True)
    d_sum = jnp.sum(emb * sgn, axis=1, keepdims=True)
    mu = (t_sum + d_sum * sgn) * (1.0 / (2 * HID))
    c = emb - mu
    c2 = c * c
    t2 = jnp.sum(c2, axis=1, keepdims=True)
    d2 = jnp.sum(c2 * sgn, axis=1, keepdims=True)
    var = (t2 + d2 * sgn) * (1.0 / (2 * HID))
    o = c * jax.lax.rsqrt(var + EPS) * gamma + beta

    out_ref[0:p, :] = o[:, 0:HID]
    out_ref[p:2 * p, :] = pltpu.roll(o, HID, 1)[:, 0:HID]


def _tc_combine(gathered, enc, age_f, abspos_f, params, n):
    grid = n // _TC_ROWS
    return pl.pallas_call(
        _combine_body,
        grid=(grid,),
        in_specs=[
            pl.BlockSpec((_TC_ROWS, 2 * HID), lambda i: (i, 0)),
            pl.BlockSpec((_TC_ROWS,), lambda i: (i,)),
            pl.BlockSpec((_TC_ROWS,), lambda i: (i,)),
            pl.BlockSpec((_TC_ROWS,), lambda i: (i,)),
            pl.BlockSpec((8, 2 * HID), lambda i: (0, 0)),
        ],
        out_specs=pl.BlockSpec((_TC_ROWS, HID), lambda i: (i, 0)),
        out_shape=jax.ShapeDtypeStruct((n, HID), jnp.float32),
        compiler_params=pltpu.CompilerParams(dimension_semantics=("parallel",)),
    )(gathered, enc, age_f, abspos_f, params)


def kernel(input_ids, token_type_ids, age, abspos, concept_table, segment_table,
           age_w0, age_phi0, age_w, age_phi,
           abspos_w0, abspos_phi0, abspos_w, abspos_phi,
           ln_gamma, ln_beta):
    b, s = input_ids.shape
    n = b * s

    w_cat_age = jnp.concatenate([age_w0.reshape(1), age_w.reshape(HID - 1)])
    phi_cat_age = jnp.concatenate([age_phi0.reshape(1), age_phi.reshape(HID - 1)])
    w_cat_abs = jnp.concatenate([abspos_w0.reshape(1), abspos_w.reshape(HID - 1)])
    phi_cat_abs = jnp.concatenate([abspos_phi0.reshape(1), abspos_phi.reshape(HID - 1)])
    params = jnp.stack([
        w_cat_age, phi_cat_age, w_cat_abs, phi_cat_abs,
        ln_gamma, ln_beta, segment_table[0], segment_table[1],
    ])
    params = jnp.concatenate([params, params], axis=1)

    ids_flat = input_ids.reshape(n)
    table_pairs = concept_table.reshape(-1, 2 * HID)
    gathered = _sc_gather(table_pairs, ids_flat >> 1, n)
    enc = (token_type_ids.reshape(n) + ((ids_flat & 1) << 1)).astype(jnp.float32)
    out = _tc_combine(
        gathered,
        enc,
        age.reshape(n),
        abspos.reshape(n),
        params,
        n,
    )
    return out.reshape(b, s, HID)
